# baseline (speedup 1.0000x reference)
import functools

import jax
import jax.numpy as jnp
from jax import lax
from jax.experimental import pallas as pl
from jax.experimental.pallas import tpu as pltpu
from jax.experimental.pallas import tpu_sc as plsc

_EPS = 1e-5
_NW = 32
_CH = 32


def kernel(xyz1, xyz2, feats1, feats2, W1, b1, g1, be1, W2, b2, g2, be2):
    B, N1, _ = xyz1.shape
    N2 = xyz2.shape[1]
    C1 = feats1.shape[1]
    C2 = feats2.shape[1]
    OC1 = W1.shape[0]
    OC2 = W2.shape[0]
    TQ = 2048
    NT = N1 // TQ
    NPTS = B * N1

    W1a = W1[:, :C2]
    W1b = W1[:, C2:]
    W2b = W2.astype(jnp.bfloat16)
    xyz1c = xyz1 - 0.5
    xyz2t = jnp.transpose(xyz2, (0, 2, 1)) - 0.5

    def k0(f2_ref, w1a_ref, g_ref):
        g_ref[0] = jax.lax.dot_general(
            f2_ref[0], w1a_ref[...], (((0,), (1,)), ((), ())),
            preferred_element_type=jnp.float32)

    G = pl.pallas_call(
        k0,
        grid=(B,),
        in_specs=[
            pl.BlockSpec((1, C2, N2), lambda b: (b, 0, 0)),
            pl.BlockSpec((OC1, C2), lambda b: (0, 0)),
        ],
        out_specs=pl.BlockSpec((1, N2, OC1), lambda b: (b, 0, 0)),
        out_shape=jax.ShapeDtypeStruct((B, N2, OC1), jnp.float32),
    )(feats2, W1a)

    def ka(xyz1_ref, xyz2t_ref, idx_ref, w_ref):
        b = pl.program_id(0)
        q = xyz1_ref[0]
        k2 = xyz2t_ref[0]
        ss = None
        for c in range(3):
            df = q[:, c:c + 1] - k2[c:c + 1, :]
            ss = df * df if ss is None else ss + df * df

        j = lax.broadcasted_iota(jnp.int32, (TQ, N2), 1)
        inff = jnp.float32(jnp.inf)
        dm = ss
        jks = []
        wks = []
        for kpass in range(3):
            mk = jnp.min(dm, axis=1, keepdims=True)
            mask = dm == mk
            ck = jnp.where(mask, j, N2)
            jks.append(jnp.min(ck, axis=1, keepdims=True))
            if kpass < 2:
                dm = jnp.where(mask, inff, dm)
            dk = jnp.maximum(jnp.sqrt(jnp.maximum(mk, 1e-16)), 1e-8)
            wks.append(1.0 / dk)
        sw = wks[0] + wks[1] + wks[2]
        idx_ref[0] = jnp.concatenate(jks, axis=1) + b * N2
        w_ref[0] = jnp.concatenate(
            [jnp.broadcast_to(wks[0] / sw, (TQ, 16)),
             jnp.broadcast_to(wks[1] / sw, (TQ, 16)),
             jnp.broadcast_to(wks[2] / sw, (TQ, 16))], axis=1)

    idx, wn = pl.pallas_call(
        ka,
        grid=(B, NT),
        in_specs=[
            pl.BlockSpec((1, TQ, 3), lambda b, t: (b, t, 0)),
            pl.BlockSpec((1, 3, N2), lambda b, t: (b, 0, 0)),
        ],
        out_specs=[
            pl.BlockSpec((1, TQ, 3), lambda b, t: (b, t, 0)),
            pl.BlockSpec((1, TQ, 48), lambda b, t: (b, t, 0)),
        ],
        out_shape=[
            jax.ShapeDtypeStruct((B, N1, 3), jnp.int32),
            jax.ShapeDtypeStruct((B, N1, 48), jnp.float32),
        ],
    )(xyz1c, xyz2t)

    idx_flat = idx.reshape(B * N1 * 3)
    w_flat = wn.reshape(B * N1 * 48)
    G2 = G.reshape(B * N2, OC1)

    QPW = NPTS // _NW
    NCH = QPW // _CH
    mesh = plsc.VectorSubcoreMesh(core_axis_name="c", subcore_axis_name="s")

    @functools.partial(
        pl.kernel,
        out_type=jax.ShapeDtypeStruct((NPTS, OC1), jnp.float32),
        mesh=mesh,
        scratch_types=[
            pltpu.VMEM((3 * _CH,), jnp.int32),
            pltpu.VMEM((48 * _CH,), jnp.float32),
            pltpu.VMEM((3 * _CH, OC1), jnp.float32),
            pltpu.VMEM((_CH, OC1), jnp.float32),
            pltpu.SemaphoreType.DMA,
        ],
    )
    def sck(g_hbm, idx_hbm, w_hbm, out_hbm, idxv, wv, rows, outv, sem):
        wid = lax.axis_index("s") * 2 + lax.axis_index("c")
        qbase = wid * QPW

        def chunk(g, carry):
            cb = qbase + g * _CH
            pltpu.sync_copy(idx_hbm.at[pl.ds(cb * 3, 3 * _CH)], idxv)
            pltpu.sync_copy(w_hbm.at[pl.ds(cb * 48, 48 * _CH)], wv)
            pltpu.async_copy(g_hbm.at[idxv], rows, sem).wait()

            def per_q(qq, c2):
                wb0 = wv[pl.ds(qq * 48, 16)]
                wb1 = wv[pl.ds(qq * 48 + 16, 16)]
                wb2 = wv[pl.ds(qq * 48 + 32, 16)]
                for c in range(OC1 // 16):
                    sl = pl.ds(16 * c, 16)
                    acc = (wb0 * rows[3 * qq, sl]
                           + wb1 * rows[3 * qq + 1, sl]
                           + wb2 * rows[3 * qq + 2, sl])
                    outv[qq, sl] = acc
                return c2

            lax.fori_loop(0, _CH, per_q, 0)
            pltpu.sync_copy(outv, out_hbm.at[pl.ds(qbase + g * _CH, _CH)])
            return carry

        lax.fori_loop(0, NCH, chunk, 0)

    x1a = sck(G2, idx_flat, w_flat)

    def kbp(x1a_ref, f1_ref, w1b_ref, x1_ref, st_ref):
        b = pl.program_id(0)
        t = pl.program_id(1)
        x1 = x1a_ref[...] + jax.lax.dot_general(
            f1_ref[0], w1b_ref[...], (((0,), (1,)), ((), ())),
            preferred_element_type=jnp.float32)
        x1_ref[0] = x1.astype(jnp.bfloat16)

        @pl.when(jnp.logical_and(b == 0, t == 0))
        def _():
            st_ref[...] = jnp.zeros_like(st_ref)

        st_ref[0:1, :] += jnp.sum(x1, axis=0, keepdims=True)
        st_ref[1:2, :] += jnp.sum(x1 * x1, axis=0, keepdims=True)

    x1, stats1 = pl.pallas_call(
        kbp,
        grid=(B, NT),
        in_specs=[
            pl.BlockSpec((TQ, OC1), lambda b, t: (b * NT + t, 0)),
            pl.BlockSpec((1, C1, TQ), lambda b, t: (b, 0, t)),
            pl.BlockSpec((OC1, C1), lambda b, t: (0, 0)),
        ],
        out_specs=[
            pl.BlockSpec((1, TQ, OC1), lambda b, t: (b, t, 0)),
            pl.BlockSpec((2, OC1), lambda b, t: (0, 0)),
        ],
        out_shape=[
            jax.ShapeDtypeStruct((B, N1, OC1), jnp.bfloat16),
            jax.ShapeDtypeStruct((2, OC1), jnp.float32),
        ],
    )(x1a, feats1, W1b)

    mean1 = stats1[0] / NPTS
    var1 = stats1[1] / NPTS - mean1 * mean1
    rstd1 = g1 / jnp.sqrt(var1 + _EPS)
    sc1 = rstd1.reshape(1, OC1)
    sh1 = (be1 - mean1 * rstd1).reshape(1, OC1)

    def kb(x1_ref, sc_ref, sh_ref, w2_ref, x2_ref, st_ref):
        b = pl.program_id(0)
        t = pl.program_id(1)
        x1f = x1_ref[0].astype(jnp.float32)
        r = jnp.maximum(x1f * sc_ref[...] + sh_ref[...], 0.0)
        x2t = jax.lax.dot_general(
            w2_ref[...], r.astype(jnp.bfloat16), (((1,), (1,)), ((), ())),
            preferred_element_type=jnp.float32)
        x2_ref[0] = x2t.astype(jnp.bfloat16)

        @pl.when(jnp.logical_and(b == 0, t == 0))
        def _():
            st_ref[...] = jnp.zeros_like(st_ref)

        st_ref[:, 0:1] += jnp.sum(x2t, axis=1, keepdims=True)
        st_ref[:, 1:2] += jnp.sum(x2t * x2t, axis=1, keepdims=True)

    x2, stats2 = pl.pallas_call(
        kb,
        grid=(B, NT),
        in_specs=[
            pl.BlockSpec((1, TQ, OC1), lambda b, t: (b, t, 0)),
            pl.BlockSpec((1, OC1), lambda b, t: (0, 0)),
            pl.BlockSpec((1, OC1), lambda b, t: (0, 0)),
            pl.BlockSpec((OC2, OC1), lambda b, t: (0, 0)),
        ],
        out_specs=[
            pl.BlockSpec((1, OC2, TQ), lambda b, t: (b, 0, t)),
            pl.BlockSpec((OC2, 2), lambda b, t: (0, 0)),
        ],
        out_shape=[
            jax.ShapeDtypeStruct((B, OC2, N1), jnp.bfloat16),
            jax.ShapeDtypeStruct((OC2, 2), jnp.float32),
        ],
    )(x1, sc1, sh1, W2b)

    mean2 = stats2[:, 0] / NPTS
    var2 = stats2[:, 1] / NPTS - mean2 * mean2
    rstd2 = g2 / jnp.sqrt(var2 + _EPS)
    sc2 = rstd2.reshape(OC2, 1)
    sh2 = (be2 - mean2 * rstd2).reshape(OC2, 1)

    def kc(x2_ref, sc_ref, sh_ref, o_ref):
        o_ref[0] = jnp.maximum(
            x2_ref[0].astype(jnp.float32) * sc_ref[...] + sh_ref[...], 0.0)

    out = pl.pallas_call(
        kc,
        grid=(B, NT),
        in_specs=[
            pl.BlockSpec((1, OC2, TQ), lambda b, t: (b, 0, t)),
            pl.BlockSpec((OC2, 1), lambda b, t: (0, 0)),
            pl.BlockSpec((OC2, 1), lambda b, t: (0, 0)),
        ],
        out_specs=pl.BlockSpec((1, OC2, TQ), lambda b, t: (b, 0, t)),
        out_shape=jax.ShapeDtypeStruct((B, OC2, N1), jnp.float32),
    )(x2, sc2, sh2)

    return out


# speedup vs baseline: 1.1795x; 1.1795x over previous
import functools

import jax
import jax.numpy as jnp
from jax import lax
from jax.experimental import pallas as pl
from jax.experimental.pallas import tpu as pltpu
from jax.experimental.pallas import tpu_sc as plsc

_EPS = 1e-5
_NW = 32
_CH = 32


def kernel(xyz1, xyz2, feats1, feats2, W1, b1, g1, be1, W2, b2, g2, be2):
    B, N1, _ = xyz1.shape
    N2 = xyz2.shape[1]
    C1 = feats1.shape[1]
    C2 = feats2.shape[1]
    OC1 = W1.shape[0]
    OC2 = W2.shape[0]
    TQ = 2048
    NT = N1 // TQ
    NPTS = B * N1

    W1a = W1[:, :C2]
    W1b = W1[:, C2:]
    W2b = W2.astype(jnp.bfloat16)
    xyz1c = xyz1 - 0.5
    xyz2t = jnp.transpose(xyz2, (0, 2, 1)) - 0.5

    def k0(f2_ref, w1a_ref, g_ref):
        g_ref[0] = jax.lax.dot_general(
            f2_ref[0], w1a_ref[...], (((0,), (1,)), ((), ())),
            preferred_element_type=jnp.float32)

    G = pl.pallas_call(
        k0,
        grid=(B,),
        in_specs=[
            pl.BlockSpec((1, C2, N2), lambda b: (b, 0, 0)),
            pl.BlockSpec((OC1, C2), lambda b: (0, 0)),
        ],
        out_specs=pl.BlockSpec((1, N2, OC1), lambda b: (b, 0, 0)),
        out_shape=jax.ShapeDtypeStruct((B, N2, OC1), jnp.float32),
    )(feats2, W1a)

    def ka(xyz1_ref, xyz2t_ref, idx_ref, w_ref):
        b = pl.program_id(0)
        q = xyz1_ref[0]
        k2 = xyz2t_ref[0]
        ss = None
        for c in range(3):
            df = q[:, c:c + 1] - k2[c:c + 1, :]
            ss = df * df if ss is None else ss + df * df

        j = lax.broadcasted_iota(jnp.int32, (TQ, N2), 1)
        inff = jnp.float32(jnp.inf)
        dm = ss
        jks = []
        wks = []
        for kpass in range(3):
            mk = jnp.min(dm, axis=1, keepdims=True)
            mask = dm == mk
            ck = jnp.where(mask, j, N2)
            jks.append(jnp.min(ck, axis=1, keepdims=True))
            if kpass < 2:
                dm = jnp.where(mask, inff, dm)
            dk = jnp.maximum(jnp.sqrt(jnp.maximum(mk, 1e-16)), 1e-8)
            wks.append(1.0 / dk)
        sw = wks[0] + wks[1] + wks[2]
        idx_ref[0] = jnp.concatenate(jks, axis=1) + b * N2
        w_ref[0] = jnp.concatenate(
            [jnp.broadcast_to(wks[0] / sw, (TQ, 16)),
             jnp.broadcast_to(wks[1] / sw, (TQ, 16)),
             jnp.broadcast_to(wks[2] / sw, (TQ, 16))], axis=1)

    idx, wn = pl.pallas_call(
        ka,
        grid=(B, NT),
        in_specs=[
            pl.BlockSpec((1, TQ, 3), lambda b, t: (b, t, 0)),
            pl.BlockSpec((1, 3, N2), lambda b, t: (b, 0, 0)),
        ],
        out_specs=[
            pl.BlockSpec((1, TQ, 3), lambda b, t: (b, t, 0)),
            pl.BlockSpec((1, TQ, 48), lambda b, t: (b, t, 0)),
        ],
        out_shape=[
            jax.ShapeDtypeStruct((B, N1, 3), jnp.int32),
            jax.ShapeDtypeStruct((B, N1, 48), jnp.float32),
        ],
    )(xyz1c, xyz2t)

    idx_flat = idx.reshape(B * N1 * 3)
    w_flat = wn.reshape(B * N1 * 48)
    G2 = G.reshape(B * N2, OC1)

    QPW = NPTS // _NW
    NCH = QPW // _CH
    mesh = plsc.VectorSubcoreMesh(core_axis_name="c", subcore_axis_name="s")

    @functools.partial(
        pl.kernel,
        out_type=jax.ShapeDtypeStruct((NPTS, OC1), jnp.float32),
        mesh=mesh,
        scratch_types=[
            pltpu.VMEM((3 * NPTS // _NW,), jnp.int32),
            pltpu.VMEM((48 * NPTS // _NW,), jnp.float32),
            pltpu.VMEM((3 * _CH, OC1), jnp.float32),
            pltpu.VMEM((3 * _CH, OC1), jnp.float32),
            pltpu.VMEM((_CH, OC1), jnp.float32),
            pltpu.SemaphoreType.DMA,
            pltpu.SemaphoreType.DMA,
        ],
    )
    def sck(g_hbm, idx_hbm, w_hbm, out_hbm, idxv, wv, rows0, rows1, outv,
            sem0, sem1):
        wid = lax.axis_index("s") * 2 + lax.axis_index("c")
        qbase = wid * QPW
        pltpu.sync_copy(idx_hbm.at[pl.ds(qbase * 3, QPW * 3)], idxv)
        pltpu.sync_copy(w_hbm.at[pl.ds(qbase * 48, QPW * 48)], wv)
        rows = (rows0, rows1)
        sems = (sem0, sem1)
        for par in range(2):
            pltpu.async_copy(
                g_hbm.at[idxv.at[pl.ds(par * 3 * _CH, 3 * _CH)]],
                rows[par], sems[par])

        def pair(h, carry):
            for par in range(2):
                g = 2 * h + par
                pltpu.make_async_copy(
                    g_hbm.at[idxv.at[pl.ds(0, 3 * _CH)]],
                    rows[par], sems[par]).wait()

                def quad(i, c2):
                    for u in range(4):
                        qq = 4 * i + u
                        ql = (g * _CH + qq) * 48
                        wb0 = wv[pl.ds(ql, 16)]
                        wb1 = wv[pl.ds(ql + 16, 16)]
                        wb2 = wv[pl.ds(ql + 32, 16)]
                        for c in range(OC1 // 16):
                            sl = pl.ds(16 * c, 16)
                            acc = (wb0 * rows[par][3 * qq, sl]
                                   + wb1 * rows[par][3 * qq + 1, sl]
                                   + wb2 * rows[par][3 * qq + 2, sl])
                            outv[qq, sl] = acc
                    return c2

                lax.fori_loop(0, _CH // 4, quad, 0)
                pltpu.sync_copy(outv,
                                out_hbm.at[pl.ds(qbase + g * _CH, _CH)])

                @pl.when(h < NCH // 2 - 1)
                def _():
                    pltpu.async_copy(
                        g_hbm.at[idxv.at[pl.ds((g + 2) * 3 * _CH, 3 * _CH)]],
                        rows[par], sems[par])
            return carry

        lax.fori_loop(0, NCH // 2, pair, 0)

    x1a = sck(G2, idx_flat, w_flat)

    def kbp(x1a_ref, f1_ref, w1b_ref, x1_ref, st_ref):
        b = pl.program_id(0)
        t = pl.program_id(1)
        x1 = x1a_ref[...] + jax.lax.dot_general(
            f1_ref[0], w1b_ref[...], (((0,), (1,)), ((), ())),
            preferred_element_type=jnp.float32)
        x1_ref[0] = x1.astype(jnp.bfloat16)

        @pl.when(jnp.logical_and(b == 0, t == 0))
        def _():
            st_ref[...] = jnp.zeros_like(st_ref)

        st_ref[0:1, :] += jnp.sum(x1, axis=0, keepdims=True)
        st_ref[1:2, :] += jnp.sum(x1 * x1, axis=0, keepdims=True)

    x1, stats1 = pl.pallas_call(
        kbp,
        grid=(B, NT),
        in_specs=[
            pl.BlockSpec((TQ, OC1), lambda b, t: (b * NT + t, 0)),
            pl.BlockSpec((1, C1, TQ), lambda b, t: (b, 0, t)),
            pl.BlockSpec((OC1, C1), lambda b, t: (0, 0)),
        ],
        out_specs=[
            pl.BlockSpec((1, TQ, OC1), lambda b, t: (b, t, 0)),
            pl.BlockSpec((2, OC1), lambda b, t: (0, 0)),
        ],
        out_shape=[
            jax.ShapeDtypeStruct((B, N1, OC1), jnp.bfloat16),
            jax.ShapeDtypeStruct((2, OC1), jnp.float32),
        ],
    )(x1a, feats1, W1b)

    mean1 = stats1[0] / NPTS
    var1 = stats1[1] / NPTS - mean1 * mean1
    rstd1 = g1 / jnp.sqrt(var1 + _EPS)
    sc1 = rstd1.reshape(1, OC1)
    sh1 = (be1 - mean1 * rstd1).reshape(1, OC1)

    def kb(x1_ref, sc_ref, sh_ref, w2_ref, x2_ref, st_ref):
        b = pl.program_id(0)
        t = pl.program_id(1)
        x1f = x1_ref[0].astype(jnp.float32)
        r = jnp.maximum(x1f * sc_ref[...] + sh_ref[...], 0.0)
        x2t = jax.lax.dot_general(
            w2_ref[...], r.astype(jnp.bfloat16), (((1,), (1,)), ((), ())),
            preferred_element_type=jnp.float32)
        x2_ref[0] = x2t.astype(jnp.bfloat16)

        @pl.when(jnp.logical_and(b == 0, t == 0))
        def _():
            st_ref[...] = jnp.zeros_like(st_ref)

        st_ref[:, 0:1] += jnp.sum(x2t, axis=1, keepdims=True)
        st_ref[:, 1:2] += jnp.sum(x2t * x2t, axis=1, keepdims=True)

    x2, stats2 = pl.pallas_call(
        kb,
        grid=(B, NT),
        in_specs=[
            pl.BlockSpec((1, TQ, OC1), lambda b, t: (b, t, 0)),
            pl.BlockSpec((1, OC1), lambda b, t: (0, 0)),
            pl.BlockSpec((1, OC1), lambda b, t: (0, 0)),
            pl.BlockSpec((OC2, OC1), lambda b, t: (0, 0)),
        ],
        out_specs=[
            pl.BlockSpec((1, OC2, TQ), lambda b, t: (b, 0, t)),
            pl.BlockSpec((OC2, 2), lambda b, t: (0, 0)),
        ],
        out_shape=[
            jax.ShapeDtypeStruct((B, OC2, N1), jnp.bfloat16),
            jax.ShapeDtypeStruct((OC2, 2), jnp.float32),
        ],
    )(x1, sc1, sh1, W2b)

    mean2 = stats2[:, 0] / NPTS
    var2 = stats2[:, 1] / NPTS - mean2 * mean2
    rstd2 = g2 / jnp.sqrt(var2 + _EPS)
    sc2 = rstd2.reshape(OC2, 1)
    sh2 = (be2 - mean2 * rstd2).reshape(OC2, 1)

    def kc(x2_ref, sc_ref, sh_ref, o_ref):
        o_ref[0] = jnp.maximum(
            x2_ref[0].astype(jnp.float32) * sc_ref[...] + sh_ref[...], 0.0)

    out = pl.pallas_call(
        kc,
        grid=(B, NT),
        in_specs=[
            pl.BlockSpec((1, OC2, TQ), lambda b, t: (b, 0, t)),
            pl.BlockSpec((OC2, 1), lambda b, t: (0, 0)),
            pl.BlockSpec((OC2, 1), lambda b, t: (0, 0)),
        ],
        out_specs=pl.BlockSpec((1, OC2, TQ), lambda b, t: (b, 0, t)),
        out_shape=jax.ShapeDtypeStruct((B, OC2, N1), jnp.float32),
    )(x2, sc2, sh2)

    return out
